# baseline (speedup 1.0000x reference)
import functools

import jax
import jax.numpy as jnp
from jax import lax
from jax.experimental import pallas as pl
from jax.experimental.pallas import tpu as pltpu
from jax.experimental.pallas import tpu_sc as plsc

_NC = 2
_NS = 16
_NW = _NC * _NS
_LANES = 16

_CH = 16


def _sc_body(B, P, D, xf, pf, of, pbuf, xbuf):
    c = lax.axis_index("c")
    s = lax.axis_index("s")
    w = s * _NC + c
    pos_per_w = P // _NW
    nch = pos_per_w // _CH
    chunk_elems = _CH * D
    pos0 = w * pos_per_w

    def chunk_body(ci, carry):
        prow = pos0 + ci * _CH
        pltpu.sync_copy(pf.at[pl.ds(prow * D, chunk_elems)], pbuf)
        for b in range(B):
            row = b * P + prow
            pltpu.sync_copy(xf.at[pl.ds(row * D, chunk_elems)], xbuf)

            @plsc.parallel_loop(0, chunk_elems, step=_LANES, unroll=8)
            def _add(i):
                xbuf[pl.ds(i, _LANES)] = (
                    xbuf[pl.ds(i, _LANES)] + pbuf[pl.ds(i, _LANES)]
                )

            pltpu.sync_copy(xbuf, of.at[pl.ds(row * D, chunk_elems)])
        return carry

    lax.fori_loop(0, nch, chunk_body, 0)


def kernel(x, pos_table):
    B, P, D = x.shape
    xf = x.reshape(B * P * D)
    pf = pos_table.reshape(P * D)
    mesh = plsc.VectorSubcoreMesh(core_axis_name="c", subcore_axis_name="s")
    k = pl.kernel(
        functools.partial(_sc_body, B, P, D),
        out_type=jax.ShapeDtypeStruct((B * P * D,), x.dtype),
        mesh=mesh,
        scratch_types=[
            pltpu.VMEM((_CH * D,), jnp.float32),
            pltpu.VMEM((_CH * D,), jnp.float32),
        ],
    )
    return k(xf, pf).reshape(B, P, D)


# speedup vs baseline: 1.2053x; 1.2053x over previous
import functools

import jax
import jax.numpy as jnp
from jax import lax
from jax.experimental import pallas as pl
from jax.experimental.pallas import tpu as pltpu
from jax.experimental.pallas import tpu_sc as plsc

_NC = 2
_NS = 16
_NW = _NC * _NS
_LANES = 16

_CH = 4


def _sc_body(B, P, D, xf, pf, of, xb, pb, sin, sp, sout):
    c = lax.axis_index("c")
    s = lax.axis_index("s")
    w = s * _NC + c
    pos_per_w = P // _NW
    nch = pos_per_w // _CH
    ce = _CH * D
    pos0 = w * pos_per_w

    def in_copy(ci, slot, b):
        start = (b * P + pos0 + ci * _CH) * D
        return pltpu.make_async_copy(xf.at[pl.ds(start, ce)], xb.at[slot, b], sin)

    def pos_copy(ci, pslot):
        start = (pos0 + ci * _CH) * D
        return pltpu.make_async_copy(pf.at[pl.ds(start, ce)], pb.at[pslot], sp)

    def out_copy(ci, slot, b):
        start = (b * P + pos0 + ci * _CH) * D
        return pltpu.make_async_copy(xb.at[slot, b], of.at[pl.ds(start, ce)], sout)

    def fire_in(ci, slot):
        for b in range(B):
            in_copy(ci, slot, b).start()

    fire_in(0, 0)
    fire_in(1, 1)
    pos_copy(0, 0).start()

    def group_body(g, carry):
        for sl in range(4):
            ci = g * 4 + sl

            @pl.when(ci + 2 < nch)
            def _():
                fire_in(ci + 2, (sl + 2) % 4)

            @pl.when(ci + 1 < nch)
            def _():
                pos_copy(ci + 1, (sl + 1) % 2).start()

            pos_copy(ci, sl % 2).wait()
            for b in range(B):
                in_copy(ci, sl, b).wait()

            @plsc.parallel_loop(0, ce, step=_LANES, unroll=4)
            def _add(i):
                pv = pb[sl % 2, pl.ds(i, _LANES)]
                for b in range(B):
                    xb[sl, b, pl.ds(i, _LANES)] = (
                        xb[sl, b, pl.ds(i, _LANES)] + pv
                    )

            for b in range(B):
                out_copy(ci, sl, b).start()

            @pl.when(ci >= 1)
            def _():
                for b in range(B):
                    out_copy(ci - 1, (sl + 3) % 4, b).wait()
        return carry

    lax.fori_loop(0, nch // 4, group_body, 0)

    for b in range(B):
        out_copy(nch - 1, (nch - 1) % 4, b).wait()


def kernel(x, pos_table):
    B, P, D = x.shape
    xf = x.reshape(B * P * D)
    pf = pos_table.reshape(P * D)
    mesh = plsc.VectorSubcoreMesh(core_axis_name="c", subcore_axis_name="s")
    k = pl.kernel(
        functools.partial(_sc_body, B, P, D),
        out_type=jax.ShapeDtypeStruct((B * P * D,), x.dtype),
        mesh=mesh,
        scratch_types=[
            pltpu.VMEM((4, B, _CH * D), jnp.float32),
            pltpu.VMEM((2, _CH * D), jnp.float32),
            pltpu.SemaphoreType.DMA,
            pltpu.SemaphoreType.DMA,
            pltpu.SemaphoreType.DMA,
        ],
    )
    return k(xf, pf).reshape(B, P, D)


# speedup vs baseline: 4.1311x; 3.4273x over previous
import functools

import jax
import jax.numpy as jnp
from jax import lax
from jax.experimental import pallas as pl
from jax.experimental.pallas import tpu as pltpu
from jax.experimental.pallas import tpu_sc as plsc

_NC = 2
_NS = 16
_NW = _NC * _NS
_LANES = 16

_CH = 8
_NSLOT = 3


def _sc_body(B, P, D, x, pt, out, xb, pb, sin, sp, sout):
    c = lax.axis_index("c")
    s = lax.axis_index("s")
    w = s * _NC + c
    pos_per_w = P // _NW
    nch = pos_per_w // _CH
    pos0 = w * pos_per_w

    def in_copy(ci, slot):
        r0 = pos0 + ci * _CH
        return pltpu.make_async_copy(x.at[:, pl.ds(r0, _CH), :], xb.at[slot], sin)

    def pos_copy(ci, ps):
        r0 = pos0 + ci * _CH
        return pltpu.make_async_copy(pt.at[pl.ds(r0, _CH), :], pb.at[ps], sp)

    def out_copy(ci, slot):
        r0 = pos0 + ci * _CH
        return pltpu.make_async_copy(xb.at[slot], out.at[:, pl.ds(r0, _CH), :], sout)

    in_copy(0, 0).start()
    in_copy(1, 1).start()
    pos_copy(0, 0).start()

    def chunk_body(ci, carry):
        slot = lax.rem(ci, _NSLOT)
        ps = lax.rem(ci, 2)

        @pl.when(ci + 1 < nch)
        def _():
            pos_copy(ci + 1, 1 - ps).start()

        pos_copy(ci, ps).wait()
        in_copy(ci, slot).wait()

        @plsc.parallel_loop(0, D, step=_LANES)
        def _add(col):
            for r in range(_CH):
                pv = pb[ps, r, pl.ds(col, _LANES)]
                for b in range(B):
                    xb[slot, b, r, pl.ds(col, _LANES)] = (
                        xb[slot, b, r, pl.ds(col, _LANES)] + pv
                    )

        out_copy(ci, slot).start()

        @pl.when(ci >= 1)
        def _():
            out_copy(ci - 1, lax.rem(ci - 1, _NSLOT)).wait()

        @pl.when(ci + 2 < nch)
        def _():
            in_copy(ci + 2, lax.rem(ci + 2, _NSLOT)).start()

        return carry

    lax.fori_loop(0, nch, chunk_body, 0)

    out_copy(nch - 1, lax.rem(nch - 1, _NSLOT)).wait()


def kernel(x, pos_table):
    B, P, D = x.shape
    mesh = plsc.VectorSubcoreMesh(core_axis_name="c", subcore_axis_name="s")
    k = pl.kernel(
        functools.partial(_sc_body, B, P, D),
        out_type=jax.ShapeDtypeStruct((B, P, D), x.dtype),
        mesh=mesh,
        scratch_types=[
            pltpu.VMEM((_NSLOT, B, _CH, D), jnp.float32),
            pltpu.VMEM((2, _CH, D), jnp.float32),
            pltpu.SemaphoreType.DMA,
            pltpu.SemaphoreType.DMA,
            pltpu.SemaphoreType.DMA,
        ],
        compiler_params=pltpu.CompilerParams(use_tc_tiling_on_sc=True),
    )
    return k(x, pos_table)
